# Initial kernel scaffold; baseline (speedup 1.0000x reference)
import jax
import jax.numpy as jnp
from jax.experimental import pallas as pl


def kernel(x, edge_index, W1l, W1r, b1, W2l, W2r, b2):
    raise NotImplementedError("write your pallas kernel here")


# speedup vs baseline: 2.4324x; 2.4324x over previous
import functools
import jax
import jax.numpy as jnp
from jax import lax
from jax.experimental import pallas as pl
from jax.experimental.pallas import tpu as pltpu
from jax.experimental.pallas import tpu_sc as plsc

N = 10000
E = 320000
D = 128

NUM_WORKERS = 32
CHUNK = 128
N_PAD = 10240
E_PAD = 327680
E_PER_W = E_PAD // NUM_WORKERS
CHUNKS_PER_W = E_PER_W // CHUNK
ROWS_PER_TILE = N_PAD // 16
WB_STEPS = ROWS_PER_TILE // CHUNK


def _seg_sum_body(y, src, dst, zrows, out, src_v, dst_v, rows_v, agg_sp, sem):
    cid = lax.axis_index("c")
    sid = lax.axis_index("s")
    wid = sid * 2 + cid
    row0 = sid * ROWS_PER_TILE

    pltpu.sync_copy(zrows, rows_v)

    def zero_body(j, carry):
        pltpu.sync_copy(rows_v, agg_sp.at[pl.ds(row0 + j * CHUNK, CHUNK)])
        return carry

    lax.fori_loop(0, WB_STEPS, zero_body, 0)
    plsc.subcore_barrier()

    ebase = wid * E_PER_W

    def edge_body(k, carry):
        base = ebase + k * CHUNK
        pltpu.sync_copy(src.at[pl.ds(base, CHUNK)], src_v)
        pltpu.sync_copy(dst.at[pl.ds(base, CHUNK)], dst_v)
        pltpu.async_copy(y.at[src_v], rows_v, sem).wait()
        pltpu.sync_copy(rows_v, agg_sp.at[dst_v], add=True)
        return carry

    lax.fori_loop(0, CHUNKS_PER_W, edge_body, 0)
    plsc.subcore_barrier()

    obase = cid * N_PAD + row0

    def wb_body(j, carry):
        pltpu.sync_copy(agg_sp.at[pl.ds(row0 + j * CHUNK, CHUNK)], rows_v)
        pltpu.sync_copy(rows_v, out.at[pl.ds(obase + j * CHUNK, CHUNK)])
        return carry

    lax.fori_loop(0, WB_STEPS, wb_body, 0)


def _cnt_body(dst, zrows, ones, out, dst_v, rows_v, ones_v, acc_sp):
    cid = lax.axis_index("c")
    sid = lax.axis_index("s")
    wid = sid * 2 + cid
    row0 = sid * ROWS_PER_TILE

    pltpu.sync_copy(zrows, rows_v)

    def zero_body(j, carry):
        pltpu.sync_copy(rows_v, acc_sp.at[pl.ds(row0 + j * CHUNK, CHUNK)])
        return carry

    lax.fori_loop(0, WB_STEPS, zero_body, 0)
    pltpu.sync_copy(ones, ones_v)
    plsc.subcore_barrier()

    ebase = wid * E_PER_W

    def edge_body(k, carry):
        base = ebase + k * CHUNK
        pltpu.sync_copy(dst.at[pl.ds(base, CHUNK)], dst_v)
        pltpu.sync_copy(ones_v, acc_sp.at[dst_v], add=True)
        return carry

    lax.fori_loop(0, CHUNKS_PER_W, edge_body, 0)
    plsc.subcore_barrier()

    obase = cid * N_PAD + row0

    def wb_body(j, carry):
        pltpu.sync_copy(acc_sp.at[pl.ds(row0 + j * CHUNK, CHUNK)], rows_v)
        pltpu.sync_copy(rows_v, out.at[pl.ds(obase + j * CHUNK, CHUNK)])
        return carry

    lax.fori_loop(0, WB_STEPS, wb_body, 0)


_sc_mesh = plsc.VectorSubcoreMesh(core_axis_name="c", subcore_axis_name="s")
_sc_out = jax.ShapeDtypeStruct((2 * N_PAD, D), jnp.float32)

_seg_sum = pl.kernel(
    _seg_sum_body,
    out_type=_sc_out,
    mesh=_sc_mesh,
    scratch_types=[
        pltpu.VMEM((CHUNK,), jnp.int32),
        pltpu.VMEM((CHUNK,), jnp.int32),
        pltpu.VMEM((CHUNK, D), jnp.float32),
        pltpu.VMEM_SHARED((N_PAD, D), jnp.float32),
        pltpu.SemaphoreType.DMA,
    ],
)

_cnt_hist = pl.kernel(
    _cnt_body,
    out_type=_sc_out,
    mesh=_sc_mesh,
    scratch_types=[
        pltpu.VMEM((CHUNK,), jnp.int32),
        pltpu.VMEM((CHUNK, D), jnp.float32),
        pltpu.VMEM((CHUNK, D), jnp.float32),
        pltpu.VMEM_SHARED((N_PAD, D), jnp.float32),
    ],
)


BM = 1024


def _mm2_body(x_ref, wl_ref, wr_ref, yl_ref, yr_ref):
    x = x_ref[...]
    yl_ref[...] = jnp.dot(x, wl_ref[...], preferred_element_type=jnp.float32)
    yr_ref[...] = jnp.dot(x, wr_ref[...], preferred_element_type=jnp.float32)


def _layer_body(aggp_ref, cntp_ref, z_ref, b_ref, wl_ref, wr_ref,
                yl_ref, yr_ref):
    agg = aggp_ref[0] + aggp_ref[1]
    cnt = cntp_ref[0, :, 0:1] + cntp_ref[1, :, 0:1]
    mean = agg / jnp.maximum(cnt, 1.0)
    h = jnp.maximum(mean + z_ref[...] + b_ref[...], 0.0)
    yl_ref[...] = jnp.dot(h, wl_ref[...], preferred_element_type=jnp.float32)
    yr_ref[...] = jnp.dot(h, wr_ref[...], preferred_element_type=jnp.float32)


def _out_body(aggp_ref, cntp_ref, z_ref, b_ref, out_ref):
    agg = aggp_ref[0] + aggp_ref[1]
    cnt = cntp_ref[0, :, 0:1] + cntp_ref[1, :, 0:1]
    out_ref[...] = agg / jnp.maximum(cnt, 1.0) + z_ref[...] + b_ref[...]


_row_spec = pl.BlockSpec((BM, D), lambda i: (i, 0))
_w_spec = pl.BlockSpec((D, D), lambda i: (0, 0))
_b_spec = pl.BlockSpec((1, D), lambda i: (0, 0))
_p_spec = pl.BlockSpec((2, BM, D), lambda i: (0, i, 0))
_GRID = (N_PAD // BM,)
_row_out = jax.ShapeDtypeStruct((N_PAD, D), jnp.float32)

_mm2 = pl.pallas_call(
    _mm2_body,
    grid=_GRID,
    in_specs=[_row_spec, _w_spec, _w_spec],
    out_specs=[_row_spec, _row_spec],
    out_shape=[_row_out, _row_out],
)

_layer = pl.pallas_call(
    _layer_body,
    grid=_GRID,
    in_specs=[_p_spec, _p_spec, _row_spec, _b_spec, _w_spec, _w_spec],
    out_specs=[_row_spec, _row_spec],
    out_shape=[_row_out, _row_out],
)

_out_comb = pl.pallas_call(
    _out_body,
    grid=_GRID,
    in_specs=[_p_spec, _p_spec, _row_spec, _b_spec],
    out_specs=_row_spec,
    out_shape=_row_out,
)


@jax.jit
def kernel(x, edge_index, W1l, W1r, b1, W2l, W2r, b2):
    src = edge_index[0]
    dst = edge_index[1]
    pad_e = E_PAD - E
    src_p = jnp.concatenate([src, jnp.full((pad_e,), N, jnp.int32)])
    dst_p = jnp.concatenate([dst, jnp.full((pad_e,), N_PAD - 1, jnp.int32)])
    x_p = jnp.pad(x, ((0, N_PAD - N), (0, 0)))
    zrows = jnp.zeros((CHUNK, D), jnp.float32)
    ones = jnp.ones((CHUNK, D), jnp.float32)
    b1r = b1.reshape(1, D)
    b2r = b2.reshape(1, D)

    cnt = _cnt_hist(dst_p, zrows, ones)
    cntp = cnt.reshape(2, N_PAD, D)
    y1, z1 = _mm2(x_p, W1l, W1r)
    agg1 = _seg_sum(y1, src_p, dst_p, zrows)
    aggp1 = agg1.reshape(2, N_PAD, D)
    y2, z2 = _layer(aggp1, cntp, z1, b1r, W2l, W2r)
    agg2 = _seg_sum(y2, src_p, dst_p, zrows)
    aggp2 = agg2.reshape(2, N_PAD, D)
    out = _out_comb(aggp2, cntp, z2, b2r)
    return out[:N]


# speedup vs baseline: 3.0075x; 1.2364x over previous
import functools
import jax
import jax.numpy as jnp
from jax import lax
from jax.experimental import pallas as pl
from jax.experimental.pallas import tpu as pltpu
from jax.experimental.pallas import tpu_sc as plsc

N = 10000
E = 320000
D = 128

NUM_WORKERS = 32
CHUNK = 128
CH = 64
NBUF = 4
N_PAD = 10240
E_PAD = 327680
E_PER_W = E_PAD // NUM_WORKERS
CHUNKS_PER_W = E_PER_W // CHUNK
CHUNKS_SS = E_PER_W // CH
ROWS_PER_TILE = N_PAD // 16
WB_STEPS = ROWS_PER_TILE // CHUNK
WB_SS = ROWS_PER_TILE // CH


def _seg_sum_body(y, src, dst, zrows, out,
                  src_v0, src_v1, src_v2, src_v3,
                  dst_v0, dst_v1, dst_v2, dst_v3,
                  rows_v0, rows_v1, rows_v2, rows_v3,
                  agg_sp,
                  sem_i0, sem_i1, sem_i2, sem_i3,
                  sem_g0, sem_g1, sem_g2, sem_g3):
    cid = lax.axis_index("c")
    sid = lax.axis_index("s")
    wid = sid * 2 + cid
    row0 = sid * ROWS_PER_TILE
    ebase = wid * E_PER_W

    sv = (src_v0, src_v1, src_v2, src_v3)
    dv = (dst_v0, dst_v1, dst_v2, dst_v3)
    rv = (rows_v0, rows_v1, rows_v2, rows_v3)
    sem_i = (sem_i0, sem_i1, sem_i2, sem_i3)
    sem_g = (sem_g0, sem_g1, sem_g2, sem_g3)

    def start_idx(k, b):
        base = ebase + k * CH
        pltpu.async_copy(src.at[pl.ds(base, CH)], sv[b], sem_i[b])
        pltpu.async_copy(dst.at[pl.ds(base, CH)], dv[b], sem_i[b])

    def wait_idx(k, b):
        base = ebase + k * CH
        pltpu.make_async_copy(src.at[pl.ds(base, CH)], sv[b], sem_i[b]).wait()
        pltpu.make_async_copy(dst.at[pl.ds(base, CH)], dv[b], sem_i[b]).wait()

    def start_gather(b):
        pltpu.async_copy(y.at[sv[b]], rv[b], sem_g[b])

    def wait_gather(b):
        pltpu.make_async_copy(y.at[sv[b]], rv[b], sem_g[b]).wait()

    def scatter(b):
        pltpu.sync_copy(rv[b], agg_sp.at[dv[b]], add=True)

    for b in range(NBUF):
        start_idx(b, b)
    for b in range(NBUF - 1):
        wait_idx(b, b)
        start_gather(b)

    pltpu.sync_copy(zrows, rows_v3)

    def zero_body(j, carry):
        pltpu.sync_copy(rows_v3, agg_sp.at[pl.ds(row0 + j * CH, CH)])
        return carry

    lax.fori_loop(0, WB_SS, zero_body, 0)
    plsc.subcore_barrier()

    def edge_body(k4, carry):
        for b in range(NBUF):
            k = NBUF * k4 + b
            wait_gather(b)
            bn = (b + NBUF - 1) % NBUF
            wait_idx(k + NBUF - 1, bn)
            start_gather(bn)
            scatter(b)
            start_idx(k + NBUF, b)
        return carry

    lax.fori_loop(0, CHUNKS_SS // NBUF - 1, edge_body, 0)

    wait_gather(0)
    wait_idx(CHUNKS_SS - 1, 3)
    start_gather(3)
    scatter(0)
    for b in range(1, NBUF):
        wait_gather(b)
        scatter(b)

    plsc.subcore_barrier()

    obase = cid * N_PAD + row0

    def wb_body(j, carry):
        pltpu.sync_copy(agg_sp.at[pl.ds(row0 + j * CH, CH)], rows_v3)
        pltpu.sync_copy(rows_v3, out.at[pl.ds(obase + j * CH, CH)])
        return carry

    lax.fori_loop(0, WB_SS, wb_body, 0)


def _cnt_body(dst, zrows, ones, out, dst_v0, dst_v1, rows_v, ones_v, acc_sp,
              sem_i0, sem_i1):
    cid = lax.axis_index("c")
    sid = lax.axis_index("s")
    wid = sid * 2 + cid
    row0 = sid * ROWS_PER_TILE
    ebase = wid * E_PER_W

    dv = (dst_v0, dst_v1)
    sem_i = (sem_i0, sem_i1)

    def start_idx(k, b):
        base = ebase + k * CHUNK
        pltpu.async_copy(dst.at[pl.ds(base, CHUNK)], dv[b], sem_i[b])

    def wait_idx(k, b):
        base = ebase + k * CHUNK
        pltpu.make_async_copy(dst.at[pl.ds(base, CHUNK)], dv[b], sem_i[b]).wait()

    start_idx(0, 0)
    start_idx(1, 1)

    pltpu.sync_copy(zrows, rows_v)

    def zero_body(j, carry):
        pltpu.sync_copy(rows_v, acc_sp.at[pl.ds(row0 + j * CHUNK, CHUNK)])
        return carry

    lax.fori_loop(0, WB_STEPS, zero_body, 0)
    pltpu.sync_copy(ones, ones_v)
    plsc.subcore_barrier()

    def edge_body(k2, carry):
        for b in (0, 1):
            k = 2 * k2 + b
            wait_idx(k, b)
            pltpu.sync_copy(ones_v, acc_sp.at[dv[b]], add=True)
            start_idx(k + 2, b)
        return carry

    lax.fori_loop(0, CHUNKS_PER_W // 2 - 1, edge_body, 0)

    wait_idx(CHUNKS_PER_W - 2, 0)
    pltpu.sync_copy(ones_v, acc_sp.at[dv[0]], add=True)
    wait_idx(CHUNKS_PER_W - 1, 1)
    pltpu.sync_copy(ones_v, acc_sp.at[dv[1]], add=True)
    plsc.subcore_barrier()

    obase = cid * N_PAD + row0

    def wb_body(j, carry):
        pltpu.sync_copy(acc_sp.at[pl.ds(row0 + j * CHUNK, CHUNK)], rows_v)
        pltpu.sync_copy(rows_v, out.at[pl.ds(obase + j * CHUNK, CHUNK)])
        return carry

    lax.fori_loop(0, WB_STEPS, wb_body, 0)


_sc_mesh = plsc.VectorSubcoreMesh(core_axis_name="c", subcore_axis_name="s")
_sc_out = jax.ShapeDtypeStruct((2 * N_PAD, D), jnp.float32)

_seg_sum = pl.kernel(
    _seg_sum_body,
    out_type=_sc_out,
    mesh=_sc_mesh,
    scratch_types=(
        [pltpu.VMEM((CH,), jnp.int32)] * 8
        + [pltpu.VMEM((CH, D), jnp.float32)] * 4
        + [pltpu.VMEM_SHARED((N_PAD, D), jnp.float32)]
        + [pltpu.SemaphoreType.DMA] * 8
    ),
)

_cnt_hist = pl.kernel(
    _cnt_body,
    out_type=_sc_out,
    mesh=_sc_mesh,
    scratch_types=[
        pltpu.VMEM((CHUNK,), jnp.int32),
        pltpu.VMEM((CHUNK,), jnp.int32),
        pltpu.VMEM((CHUNK, D), jnp.float32),
        pltpu.VMEM((CHUNK, D), jnp.float32),
        pltpu.VMEM_SHARED((N_PAD, D), jnp.float32),
        pltpu.SemaphoreType.DMA,
        pltpu.SemaphoreType.DMA,
    ],
)


BM = 1024


def _mm2_body(x_ref, wl_ref, wr_ref, yl_ref, yr_ref):
    x = x_ref[...]
    yl_ref[...] = jnp.dot(x, wl_ref[...], preferred_element_type=jnp.float32)
    yr_ref[...] = jnp.dot(x, wr_ref[...], preferred_element_type=jnp.float32)


def _layer_body(aggp_ref, cntp_ref, z_ref, b_ref, wl_ref, wr_ref,
                yl_ref, yr_ref):
    agg = aggp_ref[0] + aggp_ref[1]
    cnt = cntp_ref[0, :, 0:1] + cntp_ref[1, :, 0:1]
    mean = agg / jnp.maximum(cnt, 1.0)
    h = jnp.maximum(mean + z_ref[...] + b_ref[...], 0.0)
    yl_ref[...] = jnp.dot(h, wl_ref[...], preferred_element_type=jnp.float32)
    yr_ref[...] = jnp.dot(h, wr_ref[...], preferred_element_type=jnp.float32)


def _out_body(aggp_ref, cntp_ref, z_ref, b_ref, out_ref):
    agg = aggp_ref[0] + aggp_ref[1]
    cnt = cntp_ref[0, :, 0:1] + cntp_ref[1, :, 0:1]
    out_ref[...] = agg / jnp.maximum(cnt, 1.0) + z_ref[...] + b_ref[...]


_row_spec = pl.BlockSpec((BM, D), lambda i: (i, 0))
_w_spec = pl.BlockSpec((D, D), lambda i: (0, 0))
_b_spec = pl.BlockSpec((1, D), lambda i: (0, 0))
_p_spec = pl.BlockSpec((2, BM, D), lambda i: (0, i, 0))
_GRID = (N_PAD // BM,)
_row_out = jax.ShapeDtypeStruct((N_PAD, D), jnp.float32)

_mm2 = pl.pallas_call(
    _mm2_body,
    grid=_GRID,
    in_specs=[_row_spec, _w_spec, _w_spec],
    out_specs=[_row_spec, _row_spec],
    out_shape=[_row_out, _row_out],
)

_layer = pl.pallas_call(
    _layer_body,
    grid=_GRID,
    in_specs=[_p_spec, _p_spec, _row_spec, _b_spec, _w_spec, _w_spec],
    out_specs=[_row_spec, _row_spec],
    out_shape=[_row_out, _row_out],
)

_out_comb = pl.pallas_call(
    _out_body,
    grid=_GRID,
    in_specs=[_p_spec, _p_spec, _row_spec, _b_spec],
    out_specs=_row_spec,
    out_shape=_row_out,
)


@jax.jit
def kernel(x, edge_index, W1l, W1r, b1, W2l, W2r, b2):
    src = edge_index[0]
    dst = edge_index[1]
    pad_e = E_PAD - E
    src_p = jnp.concatenate([src, jnp.full((pad_e,), N, jnp.int32)])
    dst_p = jnp.concatenate([dst, jnp.full((pad_e,), N_PAD - 1, jnp.int32)])
    x_p = jnp.pad(x, ((0, N_PAD - N), (0, 0)))
    zrows = jnp.zeros((CHUNK, D), jnp.float32)
    zss = jnp.zeros((CH, D), jnp.float32)
    ones = jnp.ones((CHUNK, D), jnp.float32)
    b1r = b1.reshape(1, D)
    b2r = b2.reshape(1, D)

    cnt = _cnt_hist(dst_p, zrows, ones)
    cntp = cnt.reshape(2, N_PAD, D)
    y1, z1 = _mm2(x_p, W1l, W1r)
    agg1 = _seg_sum(y1, src_p, dst_p, zss)
    aggp1 = agg1.reshape(2, N_PAD, D)
    y2, z2 = _layer(aggp1, cntp, z1, b1r, W2l, W2r)
    agg2 = _seg_sum(y2, src_p, dst_p, zss)
    aggp2 = agg2.reshape(2, N_PAD, D)
    out = _out_comb(aggp2, cntp, z2, b2r)
    return out[:N]


# speedup vs baseline: 3.0241x; 1.0055x over previous
import functools
import jax
import jax.numpy as jnp
from jax import lax
from jax.experimental import pallas as pl
from jax.experimental.pallas import tpu as pltpu
from jax.experimental.pallas import tpu_sc as plsc

N = 10000
E = 320000
D = 128

NUM_WORKERS = 32
CHUNK = 128
CH = 64
NBUF = 4
N_PAD = 10240
E_PAD = 327680
E_PER_W = E_PAD // NUM_WORKERS
CHUNKS_PER_W = E_PER_W // CHUNK
CHUNKS_SS = E_PER_W // CH
ROWS_PER_TILE = N_PAD // 16
WB_STEPS = ROWS_PER_TILE // CHUNK
WB_SS = ROWS_PER_TILE // CH


def _seg_sum_body(y, src, dst, zrows, out,
                  src_v0, src_v1, src_v2, src_v3,
                  dst_v0, dst_v1, dst_v2, dst_v3,
                  rows_v0, rows_v1, rows_v2, rows_v3,
                  agg_sp,
                  sem_i0, sem_i1, sem_i2, sem_i3,
                  sem_g0, sem_g1, sem_g2, sem_g3):
    cid = lax.axis_index("c")
    sid = lax.axis_index("s")
    wid = sid * 2 + cid
    row0 = sid * ROWS_PER_TILE
    ebase = wid * E_PER_W

    sv = (src_v0, src_v1, src_v2, src_v3)
    dv = (dst_v0, dst_v1, dst_v2, dst_v3)
    rv = (rows_v0, rows_v1, rows_v2, rows_v3)
    sem_i = (sem_i0, sem_i1, sem_i2, sem_i3)
    sem_g = (sem_g0, sem_g1, sem_g2, sem_g3)

    def start_idx(k, b):
        base = ebase + k * CH
        pltpu.async_copy(src.at[pl.ds(base, CH)], sv[b], sem_i[b])
        pltpu.async_copy(dst.at[pl.ds(base, CH)], dv[b], sem_i[b])

    def wait_idx(k, b):
        base = ebase + k * CH
        pltpu.make_async_copy(src.at[pl.ds(base, CH)], sv[b], sem_i[b]).wait()
        pltpu.make_async_copy(dst.at[pl.ds(base, CH)], dv[b], sem_i[b]).wait()

    def start_gather(b):
        pltpu.async_copy(y.at[sv[b]], rv[b], sem_g[b])

    def wait_gather(b):
        pltpu.make_async_copy(y.at[sv[b]], rv[b], sem_g[b]).wait()

    def scatter(b):
        pltpu.sync_copy(rv[b], agg_sp.at[dv[b]], add=True)

    for b in range(NBUF):
        start_idx(b, b)
    for b in range(NBUF - 1):
        wait_idx(b, b)
        start_gather(b)

    pltpu.sync_copy(zrows, rows_v3)

    def zero_body(j, carry):
        pltpu.sync_copy(rows_v3, agg_sp.at[pl.ds(row0 + j * CH, CH)])
        return carry

    lax.fori_loop(0, WB_SS, zero_body, 0)
    plsc.subcore_barrier()

    def edge_body(k4, carry):
        for b in range(NBUF):
            k = NBUF * k4 + b
            wait_gather(b)
            bn = (b + NBUF - 1) % NBUF
            wait_idx(k + NBUF - 1, bn)
            start_gather(bn)
            scatter(b)
            start_idx(k + NBUF, b)
        return carry

    lax.fori_loop(0, CHUNKS_SS // NBUF - 1, edge_body, 0)

    wait_gather(0)
    wait_idx(CHUNKS_SS - 1, 3)
    start_gather(3)
    scatter(0)
    for b in range(1, NBUF):
        wait_gather(b)
        scatter(b)

    plsc.subcore_barrier()

    obase = cid * N_PAD + row0

    for j in range(WB_SS):
        bb = j % 2
        if j >= 2:
            pltpu.make_async_copy(
                rv[bb], out.at[pl.ds(obase + (j - 2) * CH, CH)],
                sem_g[bb]).wait()
        pltpu.sync_copy(agg_sp.at[pl.ds(row0 + j * CH, CH)], rv[bb])
        pltpu.async_copy(rv[bb], out.at[pl.ds(obase + j * CH, CH)], sem_g[bb])
    for j in (WB_SS - 2, WB_SS - 1):
        bb = j % 2
        pltpu.make_async_copy(
            rv[bb], out.at[pl.ds(obase + j * CH, CH)], sem_g[bb]).wait()


def _cnt_body(dst, zrows, ones, out, dst_v0, dst_v1, rows_v, ones_v, acc_sp,
              sem_i0, sem_i1):
    cid = lax.axis_index("c")
    sid = lax.axis_index("s")
    wid = sid * 2 + cid
    row0 = sid * ROWS_PER_TILE
    ebase = wid * E_PER_W

    dv = (dst_v0, dst_v1)
    sem_i = (sem_i0, sem_i1)

    def start_idx(k, b):
        base = ebase + k * CHUNK
        pltpu.async_copy(dst.at[pl.ds(base, CHUNK)], dv[b], sem_i[b])

    def wait_idx(k, b):
        base = ebase + k * CHUNK
        pltpu.make_async_copy(dst.at[pl.ds(base, CHUNK)], dv[b], sem_i[b]).wait()

    start_idx(0, 0)
    start_idx(1, 1)

    pltpu.sync_copy(zrows, rows_v)

    def zero_body(j, carry):
        pltpu.sync_copy(rows_v, acc_sp.at[pl.ds(row0 + j * CHUNK, CHUNK)])
        return carry

    lax.fori_loop(0, WB_STEPS, zero_body, 0)
    pltpu.sync_copy(ones, ones_v)
    plsc.subcore_barrier()

    def edge_body(k2, carry):
        for b in (0, 1):
            k = 2 * k2 + b
            wait_idx(k, b)
            pltpu.sync_copy(ones_v, acc_sp.at[dv[b]], add=True)
            start_idx(k + 2, b)
        return carry

    lax.fori_loop(0, CHUNKS_PER_W // 2 - 1, edge_body, 0)

    wait_idx(CHUNKS_PER_W - 2, 0)
    pltpu.sync_copy(ones_v, acc_sp.at[dv[0]], add=True)
    wait_idx(CHUNKS_PER_W - 1, 1)
    pltpu.sync_copy(ones_v, acc_sp.at[dv[1]], add=True)
    plsc.subcore_barrier()

    obase = cid * N_PAD + row0

    wv = (rows_v, ones_v)
    for j in range(WB_STEPS):
        bb = j % 2
        if j >= 2:
            pltpu.make_async_copy(
                wv[bb], out.at[pl.ds(obase + (j - 2) * CHUNK, CHUNK)],
                sem_i[bb]).wait()
        pltpu.sync_copy(acc_sp.at[pl.ds(row0 + j * CHUNK, CHUNK)], wv[bb])
        pltpu.async_copy(wv[bb], out.at[pl.ds(obase + j * CHUNK, CHUNK)],
                         sem_i[bb])
    for j in (WB_STEPS - 2, WB_STEPS - 1):
        bb = j % 2
        pltpu.make_async_copy(
            wv[bb], out.at[pl.ds(obase + j * CHUNK, CHUNK)], sem_i[bb]).wait()


_sc_mesh = plsc.VectorSubcoreMesh(core_axis_name="c", subcore_axis_name="s")
_sc_out = jax.ShapeDtypeStruct((2 * N_PAD, D), jnp.float32)

_seg_sum = pl.kernel(
    _seg_sum_body,
    out_type=_sc_out,
    mesh=_sc_mesh,
    scratch_types=(
        [pltpu.VMEM((CH,), jnp.int32)] * 8
        + [pltpu.VMEM((CH, D), jnp.float32)] * 4
        + [pltpu.VMEM_SHARED((N_PAD, D), jnp.float32)]
        + [pltpu.SemaphoreType.DMA] * 8
    ),
)

_cnt_hist = pl.kernel(
    _cnt_body,
    out_type=_sc_out,
    mesh=_sc_mesh,
    scratch_types=[
        pltpu.VMEM((CHUNK,), jnp.int32),
        pltpu.VMEM((CHUNK,), jnp.int32),
        pltpu.VMEM((CHUNK, D), jnp.float32),
        pltpu.VMEM((CHUNK, D), jnp.float32),
        pltpu.VMEM_SHARED((N_PAD, D), jnp.float32),
        pltpu.SemaphoreType.DMA,
        pltpu.SemaphoreType.DMA,
    ],
)


BM = 1024


def _mm2_body(x_ref, wl_ref, wr_ref, yl_ref, yr_ref):
    x = x_ref[...]
    yl_ref[...] = jnp.dot(x, wl_ref[...], preferred_element_type=jnp.float32)
    yr_ref[...] = jnp.dot(x, wr_ref[...], preferred_element_type=jnp.float32)


def _layer_body(aggp_ref, cntp_ref, z_ref, b_ref, wl_ref, wr_ref,
                yl_ref, yr_ref):
    agg = aggp_ref[0] + aggp_ref[1]
    cnt = cntp_ref[0, :, 0:1] + cntp_ref[1, :, 0:1]
    mean = agg / jnp.maximum(cnt, 1.0)
    h = jnp.maximum(mean + z_ref[...] + b_ref[...], 0.0)
    yl_ref[...] = jnp.dot(h, wl_ref[...], preferred_element_type=jnp.float32)
    yr_ref[...] = jnp.dot(h, wr_ref[...], preferred_element_type=jnp.float32)


def _out_body(aggp_ref, cntp_ref, z_ref, b_ref, out_ref):
    agg = aggp_ref[0] + aggp_ref[1]
    cnt = cntp_ref[0, :, 0:1] + cntp_ref[1, :, 0:1]
    out_ref[...] = agg / jnp.maximum(cnt, 1.0) + z_ref[...] + b_ref[...]


_row_spec = pl.BlockSpec((BM, D), lambda i: (i, 0))
_w_spec = pl.BlockSpec((D, D), lambda i: (0, 0))
_b_spec = pl.BlockSpec((1, D), lambda i: (0, 0))
_p_spec = pl.BlockSpec((2, BM, D), lambda i: (0, i, 0))
_GRID = (N_PAD // BM,)
_row_out = jax.ShapeDtypeStruct((N_PAD, D), jnp.float32)

_mm2 = pl.pallas_call(
    _mm2_body,
    grid=_GRID,
    in_specs=[_row_spec, _w_spec, _w_spec],
    out_specs=[_row_spec, _row_spec],
    out_shape=[_row_out, _row_out],
)

_layer = pl.pallas_call(
    _layer_body,
    grid=_GRID,
    in_specs=[_p_spec, _p_spec, _row_spec, _b_spec, _w_spec, _w_spec],
    out_specs=[_row_spec, _row_spec],
    out_shape=[_row_out, _row_out],
)

_out_comb = pl.pallas_call(
    _out_body,
    grid=_GRID,
    in_specs=[_p_spec, _p_spec, _row_spec, _b_spec],
    out_specs=_row_spec,
    out_shape=_row_out,
)


@jax.jit
def kernel(x, edge_index, W1l, W1r, b1, W2l, W2r, b2):
    src = edge_index[0]
    dst = edge_index[1]
    pad_e = E_PAD - E
    src_p = jnp.concatenate([src, jnp.full((pad_e,), N, jnp.int32)])
    dst_p = jnp.concatenate([dst, jnp.full((pad_e,), N_PAD - 1, jnp.int32)])
    x_p = jnp.pad(x, ((0, N_PAD - N), (0, 0)))
    zrows = jnp.zeros((CHUNK, D), jnp.float32)
    zss = jnp.zeros((CH, D), jnp.float32)
    ones = jnp.ones((CHUNK, D), jnp.float32)
    b1r = b1.reshape(1, D)
    b2r = b2.reshape(1, D)

    cnt = _cnt_hist(dst_p, zrows, ones)
    cntp = cnt.reshape(2, N_PAD, D)
    y1, z1 = _mm2(x_p, W1l, W1r)
    agg1 = _seg_sum(y1, src_p, dst_p, zss)
    aggp1 = agg1.reshape(2, N_PAD, D)
    y2, z2 = _layer(aggp1, cntp, z1, b1r, W2l, W2r)
    agg2 = _seg_sum(y2, src_p, dst_p, zss)
    aggp2 = agg2.reshape(2, N_PAD, D)
    out = _out_comb(aggp2, cntp, z2, b2r)
    return out[:N]


# speedup vs baseline: 3.0260x; 1.0006x over previous
import jax
import jax.numpy as jnp
from jax import lax
from jax.experimental import pallas as pl
from jax.experimental.pallas import tpu as pltpu
from jax.experimental.pallas import tpu_sc as plsc

N = 10000
E = 320000
D = 128

NUM_WORKERS = 32
CHUNK = 128
CH = 64
NBUF = 4
N_PAD = 10240
E_PAD = 327680
E_PER_W = E_PAD // NUM_WORKERS
CHUNKS_PER_W = E_PER_W // CHUNK
CHUNKS_SS = E_PER_W // CH
ROWS_PER_TILE = N_PAD // 16
WB_STEPS = ROWS_PER_TILE // CHUNK
WB_SS = ROWS_PER_TILE // CH


def _seg_sum_body(y, src, dst, zrows, out,
                  src_v0, src_v1, src_v2, src_v3,
                  dst_v0, dst_v1, dst_v2, dst_v3,
                  rows_v0, rows_v1, rows_v2, rows_v3,
                  agg_sp,
                  sem_i0, sem_i1, sem_i2, sem_i3,
                  sem_g0, sem_g1, sem_g2, sem_g3):
    cid = lax.axis_index("c")
    sid = lax.axis_index("s")
    wid = sid * 2 + cid
    row0 = sid * ROWS_PER_TILE
    ebase = wid * E_PER_W

    sv = (src_v0, src_v1, src_v2, src_v3)
    dv = (dst_v0, dst_v1, dst_v2, dst_v3)
    rv = (rows_v0, rows_v1, rows_v2, rows_v3)
    sem_i = (sem_i0, sem_i1, sem_i2, sem_i3)
    sem_g = (sem_g0, sem_g1, sem_g2, sem_g3)

    def start_idx(k, b):
        base = ebase + k * CH
        pltpu.async_copy(src.at[pl.ds(base, CH)], sv[b], sem_i[b])
        pltpu.async_copy(dst.at[pl.ds(base, CH)], dv[b], sem_i[b])

    def wait_idx(k, b):
        base = ebase + k * CH
        pltpu.make_async_copy(src.at[pl.ds(base, CH)], sv[b], sem_i[b]).wait()
        pltpu.make_async_copy(dst.at[pl.ds(base, CH)], dv[b], sem_i[b]).wait()

    def start_gather(b):
        pltpu.async_copy(y.at[sv[b]], rv[b], sem_g[b])

    def wait_gather(b):
        pltpu.make_async_copy(y.at[sv[b]], rv[b], sem_g[b]).wait()

    def scatter(b):
        pltpu.sync_copy(rv[b], agg_sp.at[dv[b]], add=True)

    for b in range(NBUF):
        start_idx(b, b)
    for b in range(NBUF - 1):
        wait_idx(b, b)
        start_gather(b)

    pltpu.sync_copy(zrows, rows_v3)

    def zero_body(j, carry):
        pltpu.sync_copy(rows_v3, agg_sp.at[pl.ds(row0 + j * CH, CH)])
        return carry

    lax.fori_loop(0, WB_SS, zero_body, 0)
    plsc.subcore_barrier()

    def edge_body(k4, carry):
        for b in range(NBUF):
            k = NBUF * k4 + b
            wait_gather(b)
            bn = (b + NBUF - 1) % NBUF
            wait_idx(k + NBUF - 1, bn)
            start_gather(bn)
            scatter(b)
            start_idx(k + NBUF, b)
        return carry

    lax.fori_loop(0, CHUNKS_SS // NBUF - 1, edge_body, 0)

    wait_gather(0)
    wait_idx(CHUNKS_SS - 1, 3)
    start_gather(3)
    scatter(0)
    for b in range(1, NBUF):
        wait_gather(b)
        scatter(b)

    plsc.subcore_barrier()

    obase = cid * N_PAD + row0

    for j in range(WB_SS):
        bb = j % 2
        if j >= 2:
            pltpu.make_async_copy(
                rv[bb], out.at[pl.ds(obase + (j - 2) * CH, CH)],
                sem_g[bb]).wait()
        pltpu.sync_copy(agg_sp.at[pl.ds(row0 + j * CH, CH)], rv[bb])
        pltpu.async_copy(rv[bb], out.at[pl.ds(obase + j * CH, CH)], sem_g[bb])
    for j in (WB_SS - 2, WB_SS - 1):
        bb = j % 2
        pltpu.make_async_copy(
            rv[bb], out.at[pl.ds(obase + j * CH, CH)], sem_g[bb]).wait()


def _cnt_body(dst, zrows, ones, out, dst_v0, dst_v1, rows_v, ones_v, acc_sp,
              sem_i0, sem_i1):
    cid = lax.axis_index("c")
    sid = lax.axis_index("s")
    wid = sid * 2 + cid
    row0 = sid * ROWS_PER_TILE
    ebase = wid * E_PER_W

    dv = (dst_v0, dst_v1)
    sem_i = (sem_i0, sem_i1)

    def start_idx(k, b):
        base = ebase + k * CHUNK
        pltpu.async_copy(dst.at[pl.ds(base, CHUNK)], dv[b], sem_i[b])

    def wait_idx(k, b):
        base = ebase + k * CHUNK
        pltpu.make_async_copy(dst.at[pl.ds(base, CHUNK)], dv[b], sem_i[b]).wait()

    start_idx(0, 0)
    start_idx(1, 1)

    pltpu.sync_copy(zrows, rows_v)

    def zero_body(j, carry):
        pltpu.sync_copy(rows_v, acc_sp.at[pl.ds(row0 + j * CHUNK, CHUNK)])
        return carry

    lax.fori_loop(0, WB_STEPS, zero_body, 0)
    pltpu.sync_copy(ones, ones_v)
    plsc.subcore_barrier()

    def edge_body(k2, carry):
        for b in (0, 1):
            k = 2 * k2 + b
            wait_idx(k, b)
            pltpu.sync_copy(ones_v, acc_sp.at[dv[b]], add=True)
            start_idx(k + 2, b)
        return carry

    lax.fori_loop(0, CHUNKS_PER_W // 2 - 1, edge_body, 0)

    wait_idx(CHUNKS_PER_W - 2, 0)
    pltpu.sync_copy(ones_v, acc_sp.at[dv[0]], add=True)
    wait_idx(CHUNKS_PER_W - 1, 1)
    pltpu.sync_copy(ones_v, acc_sp.at[dv[1]], add=True)
    plsc.subcore_barrier()

    obase = cid * N_PAD + row0

    wv = (rows_v, ones_v)
    for j in range(WB_STEPS):
        bb = j % 2
        if j >= 2:
            pltpu.make_async_copy(
                wv[bb], out.at[pl.ds(obase + (j - 2) * CHUNK, CHUNK)],
                sem_i[bb]).wait()
        pltpu.sync_copy(acc_sp.at[pl.ds(row0 + j * CHUNK, CHUNK)], wv[bb])
        pltpu.async_copy(wv[bb], out.at[pl.ds(obase + j * CHUNK, CHUNK)],
                         sem_i[bb])
    for j in (WB_STEPS - 2, WB_STEPS - 1):
        bb = j % 2
        pltpu.make_async_copy(
            wv[bb], out.at[pl.ds(obase + j * CHUNK, CHUNK)], sem_i[bb]).wait()


_sc_mesh = plsc.VectorSubcoreMesh(core_axis_name="c", subcore_axis_name="s")
_sc_out = jax.ShapeDtypeStruct((2 * N_PAD, D), jnp.float32)

_seg_sum = pl.kernel(
    _seg_sum_body,
    out_type=_sc_out,
    mesh=_sc_mesh,
    scratch_types=(
        [pltpu.VMEM((CH,), jnp.int32)] * 8
        + [pltpu.VMEM((CH, D), jnp.float32)] * 4
        + [pltpu.VMEM_SHARED((N_PAD, D), jnp.float32)]
        + [pltpu.SemaphoreType.DMA] * 8
    ),
)

_cnt_hist = pl.kernel(
    _cnt_body,
    out_type=_sc_out,
    mesh=_sc_mesh,
    scratch_types=[
        pltpu.VMEM((CHUNK,), jnp.int32),
        pltpu.VMEM((CHUNK,), jnp.int32),
        pltpu.VMEM((CHUNK, D), jnp.float32),
        pltpu.VMEM((CHUNK, D), jnp.float32),
        pltpu.VMEM_SHARED((N_PAD, D), jnp.float32),
        pltpu.SemaphoreType.DMA,
        pltpu.SemaphoreType.DMA,
    ],
)


BM = 1024


def _mm2_body(x_ref, wl_ref, wr_ref, yl_ref, yr_ref):
    x = x_ref[...]
    yl_ref[...] = jnp.dot(x, wl_ref[...], preferred_element_type=jnp.float32)
    yr_ref[...] = jnp.dot(x, wr_ref[...], preferred_element_type=jnp.float32)


def _layer_body(aggp_ref, cntp_ref, z_ref, b_ref, wl_ref, wr_ref,
                yl_ref, yr_ref):
    agg = aggp_ref[0] + aggp_ref[1]
    cnt = cntp_ref[0, :, 0:1] + cntp_ref[1, :, 0:1]
    mean = agg / jnp.maximum(cnt, 1.0)
    h = jnp.maximum(mean + z_ref[...] + b_ref[...], 0.0)
    yl_ref[...] = jnp.dot(h, wl_ref[...], preferred_element_type=jnp.float32)
    yr_ref[...] = jnp.dot(h, wr_ref[...], preferred_element_type=jnp.float32)


def _out_body(aggp_ref, cntp_ref, z_ref, b_ref, out_ref):
    agg = aggp_ref[0] + aggp_ref[1]
    cnt = cntp_ref[0, :, 0:1] + cntp_ref[1, :, 0:1]
    out_ref[...] = agg / jnp.maximum(cnt, 1.0) + z_ref[...] + b_ref[...]


_row_spec = pl.BlockSpec((BM, D), lambda i: (i, 0))
_w_spec = pl.BlockSpec((D, D), lambda i: (0, 0))
_b_spec = pl.BlockSpec((1, D), lambda i: (0, 0))
_p_spec = pl.BlockSpec((2, BM, D), lambda i: (0, i, 0))
_GRID = (N_PAD // BM,)
_row_out = jax.ShapeDtypeStruct((N_PAD, D), jnp.float32)

_mm2 = pl.pallas_call(
    _mm2_body,
    grid=_GRID,
    in_specs=[_row_spec, _w_spec, _w_spec],
    out_specs=[_row_spec, _row_spec],
    out_shape=[_row_out, _row_out],
)

_layer = pl.pallas_call(
    _layer_body,
    grid=_GRID,
    in_specs=[_p_spec, _p_spec, _row_spec, _b_spec, _w_spec, _w_spec],
    out_specs=[_row_spec, _row_spec],
    out_shape=[_row_out, _row_out],
)

_out_comb = pl.pallas_call(
    _out_body,
    grid=_GRID,
    in_specs=[_p_spec, _p_spec, _row_spec, _b_spec],
    out_specs=_row_spec,
    out_shape=_row_out,
)


@jax.jit
def kernel(x, edge_index, W1l, W1r, b1, W2l, W2r, b2):
    src = edge_index[0]
    dst = edge_index[1]
    pad_e = E_PAD - E
    src_p = jnp.concatenate([src, jnp.full((pad_e,), N, jnp.int32)])
    dst_p = jnp.concatenate([dst, jnp.full((pad_e,), N_PAD - 1, jnp.int32)])
    x_p = jnp.pad(x, ((0, N_PAD - N), (0, 0)))
    zrows = jnp.zeros((CHUNK, D), jnp.float32)
    zss = jnp.zeros((CH, D), jnp.float32)
    ones = jnp.ones((CHUNK, D), jnp.float32)
    b1r = b1.reshape(1, D)
    b2r = b2.reshape(1, D)

    cnt = _cnt_hist(dst_p, zrows, ones)
    cntp = cnt.reshape(2, N_PAD, D)
    y1, z1 = _mm2(x_p, W1l, W1r)
    agg1 = _seg_sum(y1, src_p, dst_p, zss)
    aggp1 = agg1.reshape(2, N_PAD, D)
    y2, z2 = _layer(aggp1, cntp, z1, b1r, W2l, W2r)
    agg2 = _seg_sum(y2, src_p, dst_p, zss)
    aggp2 = agg2.reshape(2, N_PAD, D)
    out = _out_comb(aggp2, cntp, z2, b2r)
    return out[:N]
